# baseline (speedup 1.0000x reference)
import jax
import jax.numpy as jnp
from jax import lax
from jax.experimental import pallas as pl
from jax.experimental.pallas import tpu as pltpu
from jax.experimental.pallas import tpu_sc as plsc

N = 100000
D = 128
S = 1024
NC = 2
NS = 16
M_TC = 23040
TCB = 256
BASE = 2408
CH = 128
ACC_ROWS = 1040
DUMMY = 1024


def _sc_body(x_hbm, ids_hbm, out_hbm, acc, zbuf, rows0, rows1, rows2, rows3,
             ids2d, sg0, sg1, sg2, sg3, ss0, ss1, ss2, ss3, semi):
    c = lax.axis_index("c")
    s = lax.axis_index("s")
    wid = c * NS + s
    row0 = M_TC + wid * BASE
    rows_w = jnp.minimum(BASE, N - row0)
    full = rows_w // CH
    tail8 = (rows_w - full * CH) // 8

    def idpre(k, carry):
        pltpu.async_copy(ids_hbm.at[pl.ds(row0 + k * CH, CH)], ids2d.at[k],
                         semi)
        return carry

    lax.fori_loop(0, full, idpre, 0)

    @pl.when(full > 0)
    def _():
        pltpu.async_copy(x_hbm.at[pl.ds(row0, CH), :], rows0, sg0)

    zeros16 = jnp.zeros((16,), jnp.float32)

    def zrow(i, carry):
        for k in range(D // 16):
            zbuf[i, pl.ds(k * 16, 16)] = zeros16
        return carry

    lax.fori_loop(0, ACC_ROWS // NS, zrow, 0)
    pltpu.sync_copy(zbuf, acc.at[pl.ds(s * (ACC_ROWS // NS), ACC_ROWS // NS)])

    dummy16 = jnp.full((16,), DUMMY, jnp.int32)
    for k in range(CH // 16):
        ids2d[full, pl.ds(k * 16, 16)] = dummy16
    t0 = row0 + full * CH

    def tailb(t, carry):
        r = t0 + t * 8
        pltpu.sync_copy(ids_hbm.at[pl.ds(r, 8)], ids2d.at[full, pl.ds(t * 8, 8)])
        pltpu.sync_copy(x_hbm.at[pl.ds(r, 8), :], rows3.at[pl.ds(t * 8, 8), :])
        return carry

    lax.fori_loop(0, tail8, tailb, 0)

    def idwait(k, carry):
        pltpu.make_async_copy(ids_hbm.at[pl.ds(row0 + k * CH, CH)],
                              ids2d.at[k], semi).wait()
        return carry

    lax.fori_loop(0, full, idwait, 0)

    plsc.subcore_barrier()

    @pl.when(tail8 > 0)
    def _():
        pltpu.sync_copy(rows3, acc.at[ids2d.at[full]], add=True)

    rows_bufs = (rows0, rows1, rows2, rows3)
    sgs = (sg0, sg1, sg2, sg3)
    sss = (ss0, ss1, ss2, ss3)

    def quad(g, carry):
        for i in range(4):
            def slot(j, b):
                rb, sg, ss = rows_bufs[b], sgs[b], sss[b]
                nb = (b + 1) & 3
                r0 = row0 + j * CH
                pltpu.make_async_copy(x_hbm.at[pl.ds(r0, CH), :], rb,
                                      sg).wait()
                pltpu.async_copy(rb, acc.at[ids2d.at[j]], ss, add=True)

                @pl.when(j >= 3)
                def _():
                    pltpu.make_async_copy(rows_bufs[nb],
                                          acc.at[ids2d.at[0]], sss[nb]).wait()

                @pl.when(j + 1 < full)
                def _():
                    pltpu.async_copy(x_hbm.at[pl.ds(r0 + CH, CH), :],
                                     rows_bufs[nb], sgs[nb])

            j = 4 * g + i
            if i == 0:
                slot(j, 0)
            else:
                @pl.when(j < full)
                def _(j=j, i=i):
                    slot(j, i)
        return carry

    lax.fori_loop(0, (full + 3) // 4, quad, 0)

    for b in range(4):
        cond = (((full - 1) % 4 == b) | ((full - 2) % 4 == b)
                | ((full - 3) % 4 == b))

        @pl.when(cond)
        def _(b=b):
            pltpu.make_async_copy(rows_bufs[b], acc.at[ids2d.at[0]],
                                  sss[b]).wait()

    plsc.subcore_barrier()

    rpt = S // NS
    pltpu.sync_copy(acc.at[pl.ds(s * rpt, rpt)],
                    out_hbm.at[c, pl.ds(s * rpt, rpt), :])


@jax.jit
def _sc_segsum(x, ids):
    mesh = plsc.VectorSubcoreMesh(core_axis_name="c", subcore_axis_name="s")
    f = pl.kernel(
        _sc_body,
        out_type=jax.ShapeDtypeStruct((NC, S, D), jnp.float32),
        mesh=mesh,
        scratch_types=[
            pltpu.VMEM_SHARED((ACC_ROWS, D), jnp.float32),
            pltpu.VMEM((ACC_ROWS // NS, D), jnp.float32),
            pltpu.VMEM((CH, D), jnp.float32),
            pltpu.VMEM((CH, D), jnp.float32),
            pltpu.VMEM((CH, D), jnp.float32),
            pltpu.VMEM((CH, D), jnp.float32),
            pltpu.VMEM((BASE // CH + 1, CH), jnp.int32),
            pltpu.SemaphoreType.DMA,
            pltpu.SemaphoreType.DMA,
            pltpu.SemaphoreType.DMA,
            pltpu.SemaphoreType.DMA,
            pltpu.SemaphoreType.DMA,
            pltpu.SemaphoreType.DMA,
            pltpu.SemaphoreType.DMA,
            pltpu.SemaphoreType.DMA,
            pltpu.SemaphoreType.DMA,
        ],
    )
    return f(x, ids)


def _tc_body(ids_ref, x_ref, o_ref):
    i = pl.program_id(0)

    @pl.when(i == 0)
    def _():
        o_ref[...] = jnp.zeros_like(o_ref)

    seg = lax.broadcasted_iota(jnp.int32, (S, TCB), 0)
    onehot_t = (seg == ids_ref[0, 0, :][None, :]).astype(jnp.float32)
    o_ref[...] += lax.dot_general(onehot_t, x_ref[...],
                                  (((1,), (0,)), ((), ())),
                                  preferred_element_type=jnp.float32)


@jax.jit
def _tc_segsum(x, ids3d):
    return pl.pallas_call(
        _tc_body,
        grid=(M_TC // TCB,),
        in_specs=[
            pl.BlockSpec((1, 1, TCB), lambda i: (i, 0, 0)),
            pl.BlockSpec((TCB, D), lambda i: (i, 0)),
        ],
        out_specs=pl.BlockSpec((S, D), lambda i: (0, 0)),
        out_shape=jax.ShapeDtypeStruct((S, D), jnp.float32),
    )(ids3d, x)


def _merge_body(p_ref, t_ref, o_ref):
    o_ref[...] = p_ref[0] + p_ref[1] + t_ref[...]


@jax.jit
def _merge(partials, tc_part):
    return pl.pallas_call(
        _merge_body,
        out_shape=jax.ShapeDtypeStruct((S, D), jnp.float32),
    )(partials, tc_part)


def kernel(x, batch_indices):
    ids = batch_indices.astype(jnp.int32)
    ids3d = ids[:M_TC].reshape(M_TC // TCB, 1, TCB)
    partials = _sc_segsum(x, ids)
    tc_part = _tc_segsum(x, ids3d)
    pooled = _merge(partials, tc_part)
    return (pooled, None)


# speedup vs baseline: 1.1796x; 1.1796x over previous
import jax
import jax.numpy as jnp
from jax import lax
from jax.experimental import pallas as pl
from jax.experimental.pallas import tpu as pltpu
from jax.experimental.pallas import tpu_sc as plsc

N = 100000
D = 128
S = 1024
NC = 2
NS = 16
M_TC = 17920
TCB = 256
BASE = 2568
CH = 128
ACC_ROWS = 1040
DUMMY = 1024


def _sc_body(x_hbm, ids_hbm, out_hbm, acc, zbuf, rows0, rows1, rows2, rows3,
             ids2d, sg0, sg1, sg2, sg3, ss0, ss1, ss2, ss3, semi):
    c = lax.axis_index("c")
    s = lax.axis_index("s")
    wid = c * NS + s
    row0 = M_TC + wid * BASE
    rows_w = jnp.minimum(BASE, N - row0)
    full = rows_w // CH
    tail8 = (rows_w - full * CH) // 8

    def idpre(k, carry):
        pltpu.async_copy(ids_hbm.at[pl.ds(row0 + k * CH, CH)], ids2d.at[k],
                         semi)
        return carry

    lax.fori_loop(0, full, idpre, 0)

    @pl.when(full > 0)
    def _():
        pltpu.async_copy(x_hbm.at[pl.ds(row0, CH), :], rows0, sg0)

    zeros16 = jnp.zeros((16,), jnp.float32)

    def zrow(i, carry):
        for k in range(D // 16):
            zbuf[i, pl.ds(k * 16, 16)] = zeros16
        return carry

    lax.fori_loop(0, ACC_ROWS // NS, zrow, 0)
    pltpu.sync_copy(zbuf, acc.at[pl.ds(s * (ACC_ROWS // NS), ACC_ROWS // NS)])

    dummy16 = jnp.full((16,), DUMMY, jnp.int32)
    for k in range(CH // 16):
        ids2d[full, pl.ds(k * 16, 16)] = dummy16
    t0 = row0 + full * CH

    def tailb(t, carry):
        r = t0 + t * 8
        pltpu.sync_copy(ids_hbm.at[pl.ds(r, 8)], ids2d.at[full, pl.ds(t * 8, 8)])
        pltpu.sync_copy(x_hbm.at[pl.ds(r, 8), :], rows3.at[pl.ds(t * 8, 8), :])
        return carry

    lax.fori_loop(0, tail8, tailb, 0)

    def idwait(k, carry):
        pltpu.make_async_copy(ids_hbm.at[pl.ds(row0 + k * CH, CH)],
                              ids2d.at[k], semi).wait()
        return carry

    lax.fori_loop(0, full, idwait, 0)

    plsc.subcore_barrier()

    @pl.when(tail8 > 0)
    def _():
        pltpu.sync_copy(rows3, acc.at[ids2d.at[full]], add=True)

    rows_bufs = (rows0, rows1, rows2, rows3)
    sgs = (sg0, sg1, sg2, sg3)
    sss = (ss0, ss1, ss2, ss3)

    def quad(g, carry):
        for i in range(4):
            def slot(j, b):
                rb, sg, ss = rows_bufs[b], sgs[b], sss[b]
                nb = (b + 1) & 3
                r0 = row0 + j * CH
                pltpu.make_async_copy(x_hbm.at[pl.ds(r0, CH), :], rb,
                                      sg).wait()
                pltpu.async_copy(rb, acc.at[ids2d.at[j]], ss, add=True)

                @pl.when(j >= 3)
                def _():
                    pltpu.make_async_copy(rows_bufs[nb],
                                          acc.at[ids2d.at[0]], sss[nb]).wait()

                @pl.when(j + 1 < full)
                def _():
                    pltpu.async_copy(x_hbm.at[pl.ds(r0 + CH, CH), :],
                                     rows_bufs[nb], sgs[nb])

            j = 4 * g + i
            if i == 0:
                slot(j, 0)
            else:
                @pl.when(j < full)
                def _(j=j, i=i):
                    slot(j, i)
        return carry

    lax.fori_loop(0, (full + 3) // 4, quad, 0)

    for b in range(4):
        cond = (((full - 1) % 4 == b) | ((full - 2) % 4 == b)
                | ((full - 3) % 4 == b))

        @pl.when(cond)
        def _(b=b):
            pltpu.make_async_copy(rows_bufs[b], acc.at[ids2d.at[0]],
                                  sss[b]).wait()

    plsc.subcore_barrier()

    rpt = S // NS
    pltpu.sync_copy(acc.at[pl.ds(s * rpt, rpt)],
                    out_hbm.at[c, pl.ds(s * rpt, rpt), :])


@jax.jit
def _sc_segsum(x, ids):
    mesh = plsc.VectorSubcoreMesh(core_axis_name="c", subcore_axis_name="s")
    f = pl.kernel(
        _sc_body,
        out_type=jax.ShapeDtypeStruct((NC, S, D), jnp.float32),
        mesh=mesh,
        scratch_types=[
            pltpu.VMEM_SHARED((ACC_ROWS, D), jnp.float32),
            pltpu.VMEM((ACC_ROWS // NS, D), jnp.float32),
            pltpu.VMEM((CH, D), jnp.float32),
            pltpu.VMEM((CH, D), jnp.float32),
            pltpu.VMEM((CH, D), jnp.float32),
            pltpu.VMEM((CH, D), jnp.float32),
            pltpu.VMEM((BASE // CH + 1, CH), jnp.int32),
            pltpu.SemaphoreType.DMA,
            pltpu.SemaphoreType.DMA,
            pltpu.SemaphoreType.DMA,
            pltpu.SemaphoreType.DMA,
            pltpu.SemaphoreType.DMA,
            pltpu.SemaphoreType.DMA,
            pltpu.SemaphoreType.DMA,
            pltpu.SemaphoreType.DMA,
            pltpu.SemaphoreType.DMA,
        ],
    )
    return f(x, ids)


def _tc_body(ids_ref, x_ref, o_ref):
    i = pl.program_id(0)

    @pl.when(i == 0)
    def _():
        o_ref[...] = jnp.zeros_like(o_ref)

    seg = lax.broadcasted_iota(jnp.int32, (S, TCB), 0)
    onehot_t = (seg == ids_ref[0, 0, :][None, :]).astype(jnp.float32)
    o_ref[...] += lax.dot_general(onehot_t, x_ref[...],
                                  (((1,), (0,)), ((), ())),
                                  preferred_element_type=jnp.float32)


@jax.jit
def _tc_segsum(x, ids3d):
    return pl.pallas_call(
        _tc_body,
        grid=(M_TC // TCB,),
        in_specs=[
            pl.BlockSpec((1, 1, TCB), lambda i: (i, 0, 0)),
            pl.BlockSpec((TCB, D), lambda i: (i, 0)),
        ],
        out_specs=pl.BlockSpec((S, D), lambda i: (0, 0)),
        out_shape=jax.ShapeDtypeStruct((S, D), jnp.float32),
    )(ids3d, x)


def _merge_body(p_ref, t_ref, o_ref):
    o_ref[...] = p_ref[0] + p_ref[1] + t_ref[...]


@jax.jit
def _merge(partials, tc_part):
    return pl.pallas_call(
        _merge_body,
        out_shape=jax.ShapeDtypeStruct((S, D), jnp.float32),
    )(partials, tc_part)


def kernel(x, batch_indices):
    ids = batch_indices.astype(jnp.int32)
    ids3d = ids[:M_TC].reshape(M_TC // TCB, 1, TCB)
    partials = _sc_segsum(x, ids)
    tc_part = _tc_segsum(x, ids3d)
    pooled = _merge(partials, tc_part)
    return (pooled, None)


# speedup vs baseline: 1.3252x; 1.1235x over previous
import jax
import jax.numpy as jnp
from jax import lax
from jax.experimental import pallas as pl
from jax.experimental.pallas import tpu as pltpu
from jax.experimental.pallas import tpu_sc as plsc

N = 100000
D = 128
S = 1024
NC = 2
NS = 16
M_TC = 14848
TCB = 256
BASE = 2688
CH = 128
ACC_ROWS = 1040
DUMMY = 1024


def _sc_body(x_hbm, ids_hbm, out_hbm, acc, zbuf, rows0, rows1, rows2, rows3,
             ids2d, sg0, sg1, sg2, sg3, ss0, ss1, ss2, ss3, semi):
    c = lax.axis_index("c")
    s = lax.axis_index("s")
    wid = c * NS + s
    row0 = M_TC + wid * BASE
    rows_w = jnp.minimum(BASE, N - row0)
    full = rows_w // CH
    tail8 = (rows_w - full * CH) // 8

    def idpre(k, carry):
        pltpu.async_copy(ids_hbm.at[pl.ds(row0 + k * CH, CH)], ids2d.at[k],
                         semi)
        return carry

    lax.fori_loop(0, full, idpre, 0)

    @pl.when(full > 0)
    def _():
        pltpu.async_copy(x_hbm.at[pl.ds(row0, CH), :], rows0, sg0)

    zeros16 = jnp.zeros((16,), jnp.float32)

    def zrow(i, carry):
        for k in range(D // 16):
            zbuf[i, pl.ds(k * 16, 16)] = zeros16
        return carry

    lax.fori_loop(0, ACC_ROWS // NS, zrow, 0)
    pltpu.sync_copy(zbuf, acc.at[pl.ds(s * (ACC_ROWS // NS), ACC_ROWS // NS)])

    dummy16 = jnp.full((16,), DUMMY, jnp.int32)
    for k in range(CH // 16):
        ids2d[full, pl.ds(k * 16, 16)] = dummy16
    t0 = row0 + full * CH

    def tailb(t, carry):
        r = t0 + t * 8
        pltpu.sync_copy(ids_hbm.at[pl.ds(r, 8)], ids2d.at[full, pl.ds(t * 8, 8)])
        pltpu.sync_copy(x_hbm.at[pl.ds(r, 8), :], rows3.at[pl.ds(t * 8, 8), :])
        return carry

    lax.fori_loop(0, tail8, tailb, 0)

    def idwait(k, carry):
        pltpu.make_async_copy(ids_hbm.at[pl.ds(row0 + k * CH, CH)],
                              ids2d.at[k], semi).wait()
        return carry

    lax.fori_loop(0, full, idwait, 0)

    plsc.subcore_barrier()

    @pl.when(tail8 > 0)
    def _():
        pltpu.sync_copy(rows3, acc.at[ids2d.at[full]], add=True)

    rows_bufs = (rows0, rows1, rows2, rows3)
    sgs = (sg0, sg1, sg2, sg3)
    sss = (ss0, ss1, ss2, ss3)

    def quad(g, carry):
        for i in range(4):
            def slot(j, b):
                rb, sg, ss = rows_bufs[b], sgs[b], sss[b]
                nb = (b + 1) & 3
                r0 = row0 + j * CH
                pltpu.make_async_copy(x_hbm.at[pl.ds(r0, CH), :], rb,
                                      sg).wait()
                pltpu.async_copy(rb, acc.at[ids2d.at[j]], ss, add=True)

                @pl.when(j >= 3)
                def _():
                    pltpu.make_async_copy(rows_bufs[nb],
                                          acc.at[ids2d.at[0]], sss[nb]).wait()

                @pl.when(j + 1 < full)
                def _():
                    pltpu.async_copy(x_hbm.at[pl.ds(r0 + CH, CH), :],
                                     rows_bufs[nb], sgs[nb])

            j = 4 * g + i
            if i == 0:
                slot(j, 0)
            else:
                @pl.when(j < full)
                def _(j=j, i=i):
                    slot(j, i)
        return carry

    lax.fori_loop(0, (full + 3) // 4, quad, 0)

    for b in range(4):
        cond = (((full - 1) % 4 == b) | ((full - 2) % 4 == b)
                | ((full - 3) % 4 == b))

        @pl.when(cond)
        def _(b=b):
            pltpu.make_async_copy(rows_bufs[b], acc.at[ids2d.at[0]],
                                  sss[b]).wait()

    plsc.subcore_barrier()

    rpt = S // NS
    pltpu.sync_copy(acc.at[pl.ds(s * rpt, rpt)],
                    out_hbm.at[c, pl.ds(s * rpt, rpt), :])


@jax.jit
def _sc_segsum(x, ids):
    mesh = plsc.VectorSubcoreMesh(core_axis_name="c", subcore_axis_name="s")
    f = pl.kernel(
        _sc_body,
        out_type=jax.ShapeDtypeStruct((NC, S, D), jnp.float32),
        mesh=mesh,
        scratch_types=[
            pltpu.VMEM_SHARED((ACC_ROWS, D), jnp.float32),
            pltpu.VMEM((ACC_ROWS // NS, D), jnp.float32),
            pltpu.VMEM((CH, D), jnp.float32),
            pltpu.VMEM((CH, D), jnp.float32),
            pltpu.VMEM((CH, D), jnp.float32),
            pltpu.VMEM((CH, D), jnp.float32),
            pltpu.VMEM((BASE // CH + 1, CH), jnp.int32),
            pltpu.SemaphoreType.DMA,
            pltpu.SemaphoreType.DMA,
            pltpu.SemaphoreType.DMA,
            pltpu.SemaphoreType.DMA,
            pltpu.SemaphoreType.DMA,
            pltpu.SemaphoreType.DMA,
            pltpu.SemaphoreType.DMA,
            pltpu.SemaphoreType.DMA,
            pltpu.SemaphoreType.DMA,
        ],
    )
    return f(x, ids)


def _tc_body(ids_ref, x_ref, o_ref):
    i = pl.program_id(0)

    @pl.when(i == 0)
    def _():
        o_ref[...] = jnp.zeros_like(o_ref)

    seg = lax.broadcasted_iota(jnp.int32, (S, TCB), 0)
    onehot_t = (seg == ids_ref[0, 0, :][None, :]).astype(jnp.float32)
    o_ref[...] += lax.dot_general(onehot_t, x_ref[...],
                                  (((1,), (0,)), ((), ())),
                                  preferred_element_type=jnp.float32)


@jax.jit
def _tc_segsum(x, ids3d):
    return pl.pallas_call(
        _tc_body,
        grid=(M_TC // TCB,),
        in_specs=[
            pl.BlockSpec((1, 1, TCB), lambda i: (i, 0, 0)),
            pl.BlockSpec((TCB, D), lambda i: (i, 0)),
        ],
        out_specs=pl.BlockSpec((S, D), lambda i: (0, 0)),
        out_shape=jax.ShapeDtypeStruct((S, D), jnp.float32),
    )(ids3d, x)


def _merge_body(p_ref, t_ref, o_ref):
    o_ref[...] = p_ref[0] + p_ref[1] + t_ref[...]


@jax.jit
def _merge(partials, tc_part):
    return pl.pallas_call(
        _merge_body,
        out_shape=jax.ShapeDtypeStruct((S, D), jnp.float32),
    )(partials, tc_part)


def kernel(x, batch_indices):
    ids = batch_indices.astype(jnp.int32)
    ids3d = ids[:M_TC].reshape(M_TC // TCB, 1, TCB)
    partials = _sc_segsum(x, ids)
    tc_part = _tc_segsum(x, ids3d)
    pooled = _merge(partials, tc_part)
    return (pooled, None)
